# Initial kernel scaffold; baseline (speedup 1.0000x reference)
import jax
import jax.numpy as jnp
from jax.experimental import pallas as pl


def kernel(self_team, opp_team, embedding, W1, b1, W2, b2):
    raise NotImplementedError("write your pallas kernel here")


# speedup vs baseline: 1.6847x; 1.6847x over previous
import functools

import jax
import jax.numpy as jnp
from jax import lax
from jax.experimental import pallas as pl
from jax.experimental.pallas import tpu as pltpu
from jax.experimental.pallas import tpu_sc as plsc

NUM_SETS = 100000
EMBED_DIM = 64
HIDDEN_DIM = 128
NUM_CLASSES = 15
BATCH = 16384
TEAM = 6

NC = 2
NS = 16
NW = NC * NS
RW = BATCH // NW
CH = 16
NCH = RW // CH
IDX_PER_CH = CH * TEAM


def _sc_pool_kernel(self_idx_hbm, opp_idx_hbm, emb_hbm,
                    out_self, out_opp,
                    idx_v, gbuf, pool_v, sem):
    wid = lax.axis_index("s") * NC + lax.axis_index("c")
    base = wid * RW

    for team_hbm, out_hbm in ((self_idx_hbm, out_self), (opp_idx_hbm, out_opp)):
        pltpu.sync_copy(team_hbm.at[wid], idx_v)

        def chunk_body(c, carry):
            cp = pltpu.async_copy(emb_hbm.at[idx_v.at[c]], gbuf, sem)
            cp.wait()
            for i in range(CH):
                for d in range(EMBED_DIM // 16):
                    sl = pl.ds(d * 16, 16)
                    s = gbuf[i * TEAM, sl]
                    for j in range(1, TEAM):
                        s = s + gbuf[i * TEAM + j, sl]
                    pool_v[c * CH + i, sl] = s
            return carry

        lax.fori_loop(0, NCH, chunk_body, 0)
        pltpu.sync_copy(pool_v, out_hbm.at[pl.ds(base, RW)])


def _sc_pool(self_idx, opp_idx, embedding):
    mesh = plsc.VectorSubcoreMesh(core_axis_name="c", subcore_axis_name="s",
                                  num_cores=NC, num_subcores=NS)
    f = functools.partial(
        pl.kernel,
        out_type=(jax.ShapeDtypeStruct((BATCH, EMBED_DIM), jnp.float32),
                  jax.ShapeDtypeStruct((BATCH, EMBED_DIM), jnp.float32)),
        mesh=mesh,
        compiler_params=pltpu.CompilerParams(use_tc_tiling_on_sc=False),
        scratch_types=[
            pltpu.VMEM((NCH, IDX_PER_CH), jnp.int32),
            pltpu.VMEM((IDX_PER_CH, EMBED_DIM), jnp.float32),
            pltpu.VMEM((RW, EMBED_DIM), jnp.float32),
            pltpu.SemaphoreType.DMA,
        ],
    )(_sc_pool_kernel)
    return f(self_idx, opp_idx, embedding)


def _mlp_kernel(ps_ref, po_ref, w1_ref, b1_ref, w2t_ref, b2_ref, out_ref):
    x = jnp.concatenate([ps_ref[...], po_ref[...]], axis=1)
    w1t = jnp.transpose(w1_ref[...]) * (1.0 / TEAM)
    h = jnp.dot(x, w1t, preferred_element_type=jnp.float32,
                precision=lax.Precision.HIGHEST) + b1_ref[...]
    h = jnp.maximum(h, 0.0)
    out_ref[...] = (jnp.dot(h, w2t_ref[...], preferred_element_type=jnp.float32,
                            precision=lax.Precision.HIGHEST)
                    + b2_ref[...])


def _mlp(pooled_self, pooled_opp, W1, b1, W2, b2):
    blk = 2048
    grid = (BATCH // blk,)
    return pl.pallas_call(
        _mlp_kernel,
        grid=grid,
        in_specs=[
            pl.BlockSpec((blk, EMBED_DIM), lambda i: (i, 0)),
            pl.BlockSpec((blk, EMBED_DIM), lambda i: (i, 0)),
            pl.BlockSpec((HIDDEN_DIM, 2 * EMBED_DIM), lambda i: (0, 0)),
            pl.BlockSpec((1, HIDDEN_DIM), lambda i: (0, 0)),
            pl.BlockSpec((HIDDEN_DIM, NUM_CLASSES), lambda i: (0, 0)),
            pl.BlockSpec((1, NUM_CLASSES), lambda i: (0, 0)),
        ],
        out_specs=pl.BlockSpec((blk, NUM_CLASSES), lambda i: (i, 0)),
        out_shape=jax.ShapeDtypeStruct((BATCH, NUM_CLASSES), jnp.float32),
    )(pooled_self, pooled_opp, W1, b1.reshape(1, HIDDEN_DIM),
      W2.T, b2.reshape(1, NUM_CLASSES))


def kernel(self_team, opp_team, embedding, W1, b1, W2, b2):
    self_idx = self_team.astype(jnp.int32).reshape(NW, NCH, IDX_PER_CH)
    opp_idx = opp_team.astype(jnp.int32).reshape(NW, NCH, IDX_PER_CH)
    pooled_self, pooled_opp = _sc_pool(self_idx, opp_idx, embedding)
    return _mlp(pooled_self, pooled_opp, W1, b1, W2, b2)


# speedup vs baseline: 1.8688x; 1.1092x over previous
import functools

import jax
import jax.numpy as jnp
from jax import lax
from jax.experimental import pallas as pl
from jax.experimental.pallas import tpu as pltpu
from jax.experimental.pallas import tpu_sc as plsc

NUM_SETS = 100000
EMBED_DIM = 64
HIDDEN_DIM = 128
NUM_CLASSES = 15
BATCH = 16384
TEAM = 6

NC = 2
NS = 16
NW = NC * NS
RW = BATCH // NW
CH = 16
NCH = RW // CH
IDX_PER_CH = CH * TEAM
IDX_PER_W = RW * TEAM


def _sc_pool_kernel(self_hbm, opp_hbm, emb_hbm, out_hbm,
                    idx_v, gbuf0, gbuf1, pool_v, sem0, sem1):
    wid = lax.axis_index("s") * NC + lax.axis_index("c")
    base = wid * RW

    def reduce_chunk(gbuf, c, col0):
        for i in range(CH):
            for d in range(EMBED_DIM // 16):
                sl = pl.ds(d * 16, 16)
                s = gbuf[i * TEAM, sl]
                for j in range(1, TEAM):
                    s = s + gbuf[i * TEAM + j, sl]
                pool_v[c * CH + i, pl.ds(col0 + d * 16, 16)] = s

    def gather_desc(c, gbuf, sem):
        return pltpu.make_async_copy(
            emb_hbm.at[idx_v.at[pl.ds(c * IDX_PER_CH, IDX_PER_CH)]], gbuf, sem)

    for t, team_hbm in enumerate((self_hbm, opp_hbm)):
        pltpu.sync_copy(team_hbm.at[wid], idx_v)
        col0 = t * EMBED_DIM

        gather_desc(0, gbuf0, sem0).start()

        def pair_body(i, carry):
            c0 = 2 * i
            gather_desc(c0 + 1, gbuf1, sem1).start()
            gather_desc(c0, gbuf0, sem0).wait()
            reduce_chunk(gbuf0, c0, col0)

            @pl.when(i < NCH // 2 - 1)
            def _():
                gather_desc(c0 + 2, gbuf0, sem0).start()

            gather_desc(c0 + 1, gbuf1, sem1).wait()
            reduce_chunk(gbuf1, c0 + 1, col0)
            return carry

        lax.fori_loop(0, NCH // 2, pair_body, 0)

    pltpu.sync_copy(pool_v, out_hbm.at[pl.ds(base, RW)])


def _sc_pool(self_idx, opp_idx, embedding):
    mesh = plsc.VectorSubcoreMesh(core_axis_name="c", subcore_axis_name="s",
                                  num_cores=NC, num_subcores=NS)
    f = functools.partial(
        pl.kernel,
        out_type=jax.ShapeDtypeStruct((BATCH, 2 * EMBED_DIM), jnp.float32),
        mesh=mesh,
        compiler_params=pltpu.CompilerParams(use_tc_tiling_on_sc=False),
        scratch_types=[
            pltpu.VMEM((IDX_PER_W,), jnp.int32),
            pltpu.VMEM((IDX_PER_CH, EMBED_DIM), jnp.float32),
            pltpu.VMEM((IDX_PER_CH, EMBED_DIM), jnp.float32),
            pltpu.VMEM((RW, 2 * EMBED_DIM), jnp.float32),
            pltpu.SemaphoreType.DMA,
            pltpu.SemaphoreType.DMA,
        ],
    )(_sc_pool_kernel)
    return f(self_idx, opp_idx, embedding)


def _mlp_kernel(x_ref, w1_ref, b1_ref, w2t_ref, b2_ref, out_ref):
    w1t = jnp.transpose(w1_ref[...]) * (1.0 / TEAM)
    h = jnp.dot(x_ref[...], w1t, preferred_element_type=jnp.float32,
                precision=lax.Precision.HIGHEST) + b1_ref[...]
    h = jnp.maximum(h, 0.0)
    out_ref[...] = (jnp.dot(h, w2t_ref[...], preferred_element_type=jnp.float32,
                            precision=lax.Precision.HIGHEST)
                    + b2_ref[...])


def _mlp(pooled, W1, b1, W2, b2):
    blk = 2048
    grid = (BATCH // blk,)
    return pl.pallas_call(
        _mlp_kernel,
        grid=grid,
        in_specs=[
            pl.BlockSpec((blk, 2 * EMBED_DIM), lambda i: (i, 0)),
            pl.BlockSpec((HIDDEN_DIM, 2 * EMBED_DIM), lambda i: (0, 0)),
            pl.BlockSpec((1, HIDDEN_DIM), lambda i: (0, 0)),
            pl.BlockSpec((HIDDEN_DIM, NUM_CLASSES), lambda i: (0, 0)),
            pl.BlockSpec((1, NUM_CLASSES), lambda i: (0, 0)),
        ],
        out_specs=pl.BlockSpec((blk, NUM_CLASSES), lambda i: (i, 0)),
        out_shape=jax.ShapeDtypeStruct((BATCH, NUM_CLASSES), jnp.float32),
    )(pooled, W1, b1.reshape(1, HIDDEN_DIM), W2.T, b2.reshape(1, NUM_CLASSES))


def kernel(self_team, opp_team, embedding, W1, b1, W2, b2):
    self_idx = self_team.astype(jnp.int32).reshape(NW, IDX_PER_W)
    opp_idx = opp_team.astype(jnp.int32).reshape(NW, IDX_PER_W)
    pooled = _sc_pool(self_idx, opp_idx, embedding)
    return _mlp(pooled, W1, b1, W2, b2)
